# Initial kernel scaffold; baseline (speedup 1.0000x reference)
import jax
import jax.numpy as jnp
from jax.experimental import pallas as pl


def kernel(params, edge_index, n_lits, n_clauses, n_vars):
    raise NotImplementedError("write your pallas kernel here")


# speedup vs baseline: 1.3861x; 1.3861x over previous
import functools

import jax
import jax.numpy as jnp
from jax import lax
from jax.experimental import pallas as pl
from jax.experimental.pallas import tpu as pltpu
from jax.experimental.pallas import tpu_sc as plsc

DIM = 128
NL = 10000
NCL = 10000
NV = 5000
E = 320000
ROUNDS = 8

NCORES = 2
NSUB = 16
NW = NCORES * NSUB
EPW = E // NW
CH = 80
NCHUNK = EPW // CH
RPA = 624


def _sig(x):
    return 1.0 / (1.0 + jnp.exp(-x))


def _lanesum16(v):
    lane = lax.iota(jnp.int32, 16)
    for sh in (8, 4, 2, 1):
        perm = jnp.bitwise_xor(lane, sh)
        v = v + v.at[perm].get(mode="promise_in_bounds")
    return v


def _edge_body(g_hbm, b_hbm, sidx_hbm, didx_hbm, w2_hbm, b2_hbm, out_hbm,
               sidx_v, didx_v, g_v, b_v, msg_v, w2_v, b2_v, agg_sh,
               sem_g, sem_b):
    c = lax.axis_index("c")
    s = lax.axis_index("s")
    wid = s * NCORES + c
    base = wid * EPW

    pltpu.sync_copy(w2_hbm, w2_v)
    pltpu.sync_copy(b2_hbm, b2_v)

    def _zrow(i, _):
        for j in range(8):
            msg_v[i, pl.ds(16 * j, 16)] = jnp.zeros((16,), jnp.float32)
        return ()
    lax.fori_loop(0, CH, _zrow, ())

    zbase = s * RPA
    for k in range(8):
        pltpu.sync_copy(msg_v, agg_sh.at[pl.ds(zbase + CH * k, CH)])
    plsc.subcore_barrier()

    def _chunk(k, _):
        off = base + k * CH
        pltpu.sync_copy(sidx_hbm.at[pl.ds(off, CH)], sidx_v)
        pltpu.sync_copy(didx_hbm.at[pl.ds(off, CH)], didx_v)
        cp_g = pltpu.async_copy(g_hbm.at[sidx_v], g_v, sem_g)
        cp_b = pltpu.async_copy(b_hbm.at[didx_v], b_v, sem_b)
        cp_g.wait()
        cp_b.wait()

        def _edge(e, _):
            acc = jnp.zeros((16,), jnp.float32)
            for j in range(8):
                a = g_v[e, pl.ds(16 * j, 16)]
                bb = b_v[e, pl.ds(16 * j, 16)]
                h = jnp.maximum(a + bb, 0.0)
                acc = acc + h * w2_v[pl.ds(16 * j, 16)]
            aw = _sig(_lanesum16(acc) + b2_v[...])
            for j in range(8):
                m = g_v[e, pl.ds(128 + 16 * j, 16)]
                msg_v[e, pl.ds(16 * j, 16)] = m * aw
            return ()
        lax.fori_loop(0, CH, _edge, ())

        pltpu.sync_copy(msg_v, agg_sh.at[didx_v], add=True)
        return ()
    lax.fori_loop(0, NCHUNK, _chunk, ())

    plsc.subcore_barrier()

    @pl.when(s < NSUB - 1)
    def _():
        pltpu.sync_copy(agg_sh.at[pl.ds(s * RPA, RPA)],
                        out_hbm.at[c, pl.ds(s * RPA, RPA)])

    @pl.when(s == NSUB - 1)
    def _():
        last = NL - (NSUB - 1) * RPA
        pltpu.sync_copy(agg_sh.at[pl.ds((NSUB - 1) * RPA, last)],
                        out_hbm.at[c, pl.ds((NSUB - 1) * RPA, last)])


_EDGE_SC_CACHE = []


def _edge_sc(*args):
    if not _EDGE_SC_CACHE:
        _EDGE_SC_CACHE.append(functools.partial(
            pl.kernel,
            mesh=plsc.VectorSubcoreMesh(core_axis_name="c", subcore_axis_name="s"),
            out_type=jax.ShapeDtypeStruct((NCORES, NL, DIM), jnp.float32),
            scratch_types=[
                pltpu.VMEM((CH,), jnp.int32),
                pltpu.VMEM((CH,), jnp.int32),
                pltpu.VMEM((CH, 2 * DIM), jnp.float32),
                pltpu.VMEM((CH, DIM), jnp.float32),
                pltpu.VMEM((CH, DIM), jnp.float32),
                pltpu.VMEM((DIM,), jnp.float32),
                pltpu.VMEM((16,), jnp.float32),
                pltpu.VMEM_SHARED((NL, DIM), jnp.float32),
                pltpu.SemaphoreType.DMA,
                pltpu.SemaphoreType.DMA,
            ],
        )(_edge_body))
    return _EDGE_SC_CACHE[0](*args)


def _prep_body(src_ref, dst_ref, was_ref, wad_ref, ba_ref,
               w0_ref, b0_ref, w1_ref, b1_ref, w2_ref, b2_ref,
               g_ref, b_ref):
    src = src_ref[...]
    a = jnp.dot(src, was_ref[...], preferred_element_type=jnp.float32)
    b_ref[...] = (jnp.dot(dst_ref[...], wad_ref[...],
                          preferred_element_type=jnp.float32) + ba_ref[...])
    h = jnp.maximum(jnp.dot(src, w0_ref[...],
                            preferred_element_type=jnp.float32) + b0_ref[...], 0.0)
    h = jnp.maximum(jnp.dot(h, w1_ref[...],
                            preferred_element_type=jnp.float32) + b1_ref[...], 0.0)
    m = jnp.dot(h, w2_ref[...], preferred_element_type=jnp.float32) + b2_ref[...]
    g_ref[...] = jnp.concatenate([a, m], axis=1)


def _prep(src_h, dst_h, was, wad, ba, w0, b0, w1, b1, w2, b2):
    return pl.pallas_call(
        _prep_body,
        out_shape=[jax.ShapeDtypeStruct((NL, 2 * DIM), jnp.float32),
                   jax.ShapeDtypeStruct((NL, DIM), jnp.float32)],
    )(src_h, dst_h, was, wad, ba, w0, b0, w1, b1, w2, b2)


def _lstm_bn(p0, p1, h, c, wih, whh, b, gamma, beta, extra):
    x_in = p0 + p1
    if extra is not None:
        x_in = jnp.concatenate([x_in, extra], axis=1)
    g = (jnp.dot(x_in, wih, preferred_element_type=jnp.float32)
         + jnp.dot(h, whh, preferred_element_type=jnp.float32) + b)
    i = _sig(g[:, :DIM])
    f = _sig(g[:, DIM:2 * DIM])
    gg = jnp.tanh(g[:, 2 * DIM:3 * DIM])
    o = _sig(g[:, 3 * DIM:])
    c2 = f * c + i * gg
    h2 = o * jnp.tanh(c2)
    x = h + h2
    mu = jnp.mean(x, axis=0, keepdims=True)
    var = jnp.mean((x - mu) ** 2, axis=0, keepdims=True)
    hn = (x - mu) / jnp.sqrt(var + 1e-5) * gamma + beta
    return hn, c2


def _upd_clause_body(p0_ref, p1_ref, h_ref, c_ref, wih_ref, whh_ref, b_ref,
                     gam_ref, bet_ref, oh_ref, oc_ref):
    hn, c2 = _lstm_bn(p0_ref[...], p1_ref[...], h_ref[...], c_ref[...],
                      wih_ref[...], whh_ref[...], b_ref[...],
                      gam_ref[...], bet_ref[...], None)
    oh_ref[...] = hn
    oc_ref[...] = c2


def _upd_clause(p0, p1, h, c, wih, whh, b, gamma, beta):
    return pl.pallas_call(
        _upd_clause_body,
        out_shape=[jax.ShapeDtypeStruct((NCL, DIM), jnp.float32),
                   jax.ShapeDtypeStruct((NCL, DIM), jnp.float32)],
    )(p0, p1, h, c, wih, whh, b, gamma, beta)


def _upd_lit_body(p0_ref, p1_ref, h_ref, c_ref, prev_ref,
                  wih_ref, whh_ref, b_ref, gam_ref, bet_ref,
                  oh_ref, oc_ref, chg_ref):
    h = h_ref[...]
    flipped = jnp.concatenate([h[NV:], h[:NV]], axis=0)
    hn, c2 = _lstm_bn(p0_ref[...], p1_ref[...], h, c_ref[...],
                      wih_ref[...], whh_ref[...], b_ref[...],
                      gam_ref[...], bet_ref[...], flipped)
    oh_ref[...] = hn
    oc_ref[...] = c2
    prev = prev_ref[...]
    num = jnp.sqrt(jnp.sum((hn - prev) ** 2))
    den = jnp.sqrt(jnp.sum(prev ** 2)) + 1e-6
    chg_ref[...] = (num / den).reshape(1, 1)


def _upd_lit(p0, p1, h, c, prev, wih, whh, b, gamma, beta):
    return pl.pallas_call(
        _upd_lit_body,
        out_shape=[jax.ShapeDtypeStruct((NL, DIM), jnp.float32),
                   jax.ShapeDtypeStruct((NL, DIM), jnp.float32),
                   jax.ShapeDtypeStruct((1, 1), jnp.float32)],
    )(p0, p1, h, c, prev, wih, whh, b, gamma, beta)


def _vote_body(h_ref, w0_ref, b0_ref, w1_ref, b1_ref, w2r_ref, b2_ref,
               out_ref):
    h = jnp.maximum(jnp.dot(h_ref[...], w0_ref[...],
                            preferred_element_type=jnp.float32) + b0_ref[...], 0.0)
    h = jnp.maximum(jnp.dot(h, w1_ref[...],
                            preferred_element_type=jnp.float32) + b1_ref[...], 0.0)
    votes = jnp.sum(h * w2r_ref[...], axis=1)
    pred = jnp.mean(votes) + b2_ref[0, 0]
    out_ref[...] = _sig(pred).reshape(1, 1)


def _vote(h, w0, b0, w1, b1, w2r, b2):
    return pl.pallas_call(
        _vote_body,
        out_shape=jax.ShapeDtypeStruct((1, 1), jnp.float32),
    )(h, w0, b0, w1, b1, w2r, b2)


def kernel(params, edge_index, n_lits, n_clauses, n_vars):
    p = params
    f32 = jnp.float32
    lit_idx = edge_index[0].astype(jnp.int32)
    cl_idx = edge_index[1].astype(jnp.int32)

    def lin_t(wb):
        w, b = wb
        return w.T.astype(f32), b.reshape(1, -1).astype(f32)

    def attn_parts(ps):
        (w1, b1), (w2, b2) = ps
        was = w1[:, :DIM].T.astype(f32)
        wad = w1[:, DIM:].T.astype(f32)
        ba = b1.reshape(1, -1).astype(f32)
        w2v = w2[0].astype(f32)
        b2v = jnp.full((16,), b2[0], f32)
        return was, wad, ba, w2v, b2v

    la_was, la_wad, la_ba, la_w2, la_b2 = attn_parts(p['lit_attn'])
    ca_was, ca_wad, ca_ba, ca_w2, ca_b2 = attn_parts(p['clause_attn'])
    lm = [lin_t(wb) for wb in p['lit_msg']]
    cm = [lin_t(wb) for wb in p['clause_msg']]

    def lstm_t(ps):
        wih, whh, bih, bhh = ps
        return (wih.T.astype(f32), whh.T.astype(f32),
                (bih + bhh).reshape(1, -1).astype(f32))
    l_wih, l_whh, l_b = lstm_t(p['lit_lstm'])
    c_wih, c_whh, c_b = lstm_t(p['clause_lstm'])
    l_gam = p['lit_bn'][0].reshape(1, -1).astype(f32)
    l_bet = p['lit_bn'][1].reshape(1, -1).astype(f32)
    c_gam = p['clause_bn'][0].reshape(1, -1).astype(f32)
    c_bet = p['clause_bn'][1].reshape(1, -1).astype(f32)
    v0 = lin_t(p['vote'][0])
    v1 = lin_t(p['vote'][1])
    v_w2r = p['vote'][2][0].reshape(1, -1).astype(f32)
    v_b2 = p['vote'][2][1].reshape(1, 1).astype(f32)

    sizes_zero = 0.0 * (jnp.asarray(n_lits) + jnp.asarray(n_clauses)
                        + jnp.asarray(n_vars)).astype(f32)
    lit_row = (p['lit_init'][0][:, 0] + p['lit_init'][1]).astype(f32) + sizes_zero
    cl_row = (p['clause_init'][0][:, 0] + p['clause_init'][1]).astype(f32)
    lit_h = jnp.broadcast_to(lit_row, (NL, DIM))
    clause_h = jnp.broadcast_to(cl_row, (NCL, DIM))
    lit_c = jnp.zeros((NL, DIM), f32)
    clause_c = jnp.zeros((NCL, DIM), f32)
    prev_lit_h = lit_h
    done = jnp.array(False)

    for _ in range(ROUNDS):
        g1, b1 = _prep(lit_h, clause_h, la_was, la_wad, la_ba,
                       lm[0][0], lm[0][1], lm[1][0], lm[1][1], lm[2][0], lm[2][1])
        parts1 = _edge_sc(g1, b1, lit_idx, cl_idx, la_w2, la_b2)
        clause_h_new, clause_c_new = _upd_clause(
            parts1[0], parts1[1], clause_h, clause_c,
            c_wih, c_whh, c_b, c_gam, c_bet)
        g2, b2 = _prep(clause_h_new, lit_h, ca_was, ca_wad, ca_ba,
                       cm[0][0], cm[0][1], cm[1][0], cm[1][1], cm[2][0], cm[2][1])
        parts2 = _edge_sc(g2, b2, cl_idx, lit_idx, ca_w2, ca_b2)
        lit_h_new, lit_c_new, change = _upd_lit(
            parts2[0], parts2[1], lit_h, lit_c, prev_lit_h,
            l_wih, l_whh, l_b, l_gam, l_bet)

        clause_h = jnp.where(done, clause_h, clause_h_new)
        clause_c = jnp.where(done, clause_c, clause_c_new)
        lit_c = jnp.where(done, lit_c, lit_c_new)
        lit_h = jnp.where(done, lit_h, lit_h_new)
        prev_lit_h = jnp.where(done, prev_lit_h, lit_h_new)
        done = done | (change[0, 0] < 0.01)

    return _vote(lit_h, v0[0], v0[1], v1[0], v1[1], v_w2r, v_b2)


# speedup vs baseline: 2.9738x; 2.1454x over previous
import functools

import jax
import jax.numpy as jnp
from jax import lax
from jax.experimental import pallas as pl
from jax.experimental.pallas import tpu as pltpu
from jax.experimental.pallas import tpu_sc as plsc

DIM = 128
NL = 10000
NCL = 10000
NV = 5000
E = 320000
ROUNDS = 8

NCORES = 2
NSUB = 16
NW = NCORES * NSUB
EPW = E // NW
CH = 40
NCHUNK = EPW // CH
RPA = 624


def _sig(x):
    return 1.0 / (1.0 + jnp.exp(-x))


def _lanesum16(v):
    lane = lax.iota(jnp.int32, 16)
    for sh in (8, 4, 2, 1):
        perm = jnp.bitwise_xor(lane, sh)
        v = v + v.at[perm].get(mode="promise_in_bounds")
    return v


def _edge_body(g_hbm, b_hbm, sidx_hbm, didx_hbm, w2_hbm, b2_hbm, out_hbm,
               sidx_sc0, sidx_sc1, didx_sc0, didx_sc1, g_v0, g_v1, b_v0, b_v1,
               msg_v0, w2_v, b2_v, agg_sh,
               sem_g0, sem_g1, sem_b0, sem_b1, sem_i0, sem_i1):
    c = lax.axis_index("c")
    s = lax.axis_index("s")
    wid = s * NCORES + c
    base = wid * EPW

    didx_sc = (didx_sc0, didx_sc1)
    g_v = (g_v0, g_v1)
    b_v = (b_v0, b_v1)
    sem_g = (sem_g0, sem_g1)
    sem_b = (sem_b0, sem_b1)
    sem_i = (sem_i0, sem_i1)

    sidx_sc = (sidx_sc0, sidx_sc1)

    pltpu.sync_copy(w2_hbm, w2_v)
    pltpu.sync_copy(b2_hbm, b2_v)

    def _zrow(i, _):
        for j in range(8):
            msg_v0[i, pl.ds(16 * j, 16)] = jnp.zeros((16,), jnp.float32)
        return ()
    lax.fori_loop(0, CH, _zrow, ())
    zbase = s * RPA
    for k in range(16):
        pltpu.sync_copy(msg_v0, agg_sh.at[pl.ds(zbase + CH * k, CH)])
    plsc.subcore_barrier()

    def _wait(dummy_hbm, dst, sem):
        pltpu.make_async_copy(dummy_hbm, dst, sem).wait()

    def _process(k, p):
        _wait(g_hbm.at[pl.ds(0, CH)], g_v[p], sem_g[p])
        _wait(b_hbm.at[pl.ds(0, CH)], b_v[p], sem_b[p])
        gp, bp, mp = g_v[p], b_v[p], msg_v0

        @plsc.parallel_loop(0, CH, unroll=4)
        def _edge(e):
            acc = jnp.zeros((16,), jnp.float32)
            for j in range(8):
                a = gp[e, pl.ds(16 * j, 16)]
                bb = bp[e, pl.ds(16 * j, 16)]
                h = jnp.maximum(a + bb, 0.0)
                acc = acc + h * w2_v[pl.ds(16 * j, 16)]
            aw = _sig(_lanesum16(acc) + b2_v[...])
            for j in range(8):
                m = gp[e, pl.ds(128 + 16 * j, 16)]
                mp[e, pl.ds(16 * j, 16)] = m * aw

        pltpu.sync_copy(mp, agg_sh.at[didx_sc[p]], add=True)

    def _issue_full(k, p):
        off = base + k * CH
        pltpu.async_copy(didx_hbm.at[pl.ds(off, CH)], didx_sc[p], sem_i[p])
        pltpu.async_copy(sidx_hbm.at[pl.ds(off, CH)], sidx_sc[p], sem_i[p])
        _wait(didx_hbm.at[pl.ds(0, CH)], didx_sc[p], sem_i[p])
        _wait(sidx_hbm.at[pl.ds(0, CH)], sidx_sc[p], sem_i[p])
        pltpu.async_copy(g_hbm.at[sidx_sc[p]], g_v[p], sem_g[p])
        pltpu.async_copy(b_hbm.at[didx_sc[p]], b_v[p], sem_b[p])

    _issue_full(0, 0)
    _issue_full(1, 1)

    def _pair(k2, _):
        for p in range(2):
            k = 2 * k2 + p

            @pl.when(k < NCHUNK)
            def _():
                _process(k, p)

                @pl.when(k + 2 < NCHUNK)
                def _():
                    _issue_full(k + 2, p)
        return ()
    lax.fori_loop(0, (NCHUNK + 1) // 2, _pair, ())

    plsc.subcore_barrier()

    @pl.when(s < NSUB - 1)
    def _():
        pltpu.sync_copy(agg_sh.at[pl.ds(s * RPA, RPA)],
                        out_hbm.at[c, pl.ds(s * RPA, RPA)])

    @pl.when(s == NSUB - 1)
    def _():
        last = NL - (NSUB - 1) * RPA
        pltpu.sync_copy(agg_sh.at[pl.ds((NSUB - 1) * RPA, last)],
                        out_hbm.at[c, pl.ds((NSUB - 1) * RPA, last)])


_EDGE_SC_CACHE = []


def _edge_sc(*args):
    if not _EDGE_SC_CACHE:
        _EDGE_SC_CACHE.append(functools.partial(
            pl.kernel,
            mesh=plsc.VectorSubcoreMesh(core_axis_name="c", subcore_axis_name="s"),
            out_type=jax.ShapeDtypeStruct((NCORES, NL, DIM), jnp.float32),
            scratch_types=[
                pltpu.VMEM((CH,), jnp.int32),
                pltpu.VMEM((CH,), jnp.int32),
                pltpu.VMEM((CH,), jnp.int32),
                pltpu.VMEM((CH,), jnp.int32),
                pltpu.VMEM((CH, 2 * DIM), jnp.float32),
                pltpu.VMEM((CH, 2 * DIM), jnp.float32),
                pltpu.VMEM((CH, DIM), jnp.float32),
                pltpu.VMEM((CH, DIM), jnp.float32),
                pltpu.VMEM((CH, DIM), jnp.float32),
                pltpu.VMEM((DIM,), jnp.float32),
                pltpu.VMEM((16,), jnp.float32),
                pltpu.VMEM_SHARED((NL, DIM), jnp.float32),
                pltpu.SemaphoreType.DMA,
                pltpu.SemaphoreType.DMA,
                pltpu.SemaphoreType.DMA,
                pltpu.SemaphoreType.DMA,
                pltpu.SemaphoreType.DMA,
                pltpu.SemaphoreType.DMA,
            ],
        )(_edge_body))
    return _EDGE_SC_CACHE[0](*args)


def _prep_body(src_ref, dst_ref, was_ref, wad_ref, ba_ref,
               w0_ref, b0_ref, w1_ref, b1_ref, w2_ref, b2_ref,
               g_ref, b_ref):
    src = src_ref[...]
    a = jnp.dot(src, was_ref[...], preferred_element_type=jnp.float32)
    b_ref[...] = (jnp.dot(dst_ref[...], wad_ref[...],
                          preferred_element_type=jnp.float32) + ba_ref[...])
    h = jnp.maximum(jnp.dot(src, w0_ref[...],
                            preferred_element_type=jnp.float32) + b0_ref[...], 0.0)
    h = jnp.maximum(jnp.dot(h, w1_ref[...],
                            preferred_element_type=jnp.float32) + b1_ref[...], 0.0)
    m = jnp.dot(h, w2_ref[...], preferred_element_type=jnp.float32) + b2_ref[...]
    g_ref[...] = jnp.concatenate([a, m], axis=1)


def _prep(src_h, dst_h, was, wad, ba, w0, b0, w1, b1, w2, b2):
    return pl.pallas_call(
        _prep_body,
        out_shape=[jax.ShapeDtypeStruct((NL, 2 * DIM), jnp.float32),
                   jax.ShapeDtypeStruct((NL, DIM), jnp.float32)],
    )(src_h, dst_h, was, wad, ba, w0, b0, w1, b1, w2, b2)


def _lstm_bn(p0, p1, h, c, wih, whh, b, gamma, beta, extra):
    x_in = p0 + p1
    if extra is not None:
        x_in = jnp.concatenate([x_in, extra], axis=1)
    g = (jnp.dot(x_in, wih, preferred_element_type=jnp.float32)
         + jnp.dot(h, whh, preferred_element_type=jnp.float32) + b)
    i = _sig(g[:, :DIM])
    f = _sig(g[:, DIM:2 * DIM])
    gg = jnp.tanh(g[:, 2 * DIM:3 * DIM])
    o = _sig(g[:, 3 * DIM:])
    c2 = f * c + i * gg
    h2 = o * jnp.tanh(c2)
    x = h + h2
    mu = jnp.mean(x, axis=0, keepdims=True)
    var = jnp.mean((x - mu) ** 2, axis=0, keepdims=True)
    hn = (x - mu) / jnp.sqrt(var + 1e-5) * gamma + beta
    return hn, c2


def _upd_clause_body(p0_ref, p1_ref, h_ref, c_ref, wih_ref, whh_ref, b_ref,
                     gam_ref, bet_ref, oh_ref, oc_ref):
    hn, c2 = _lstm_bn(p0_ref[...], p1_ref[...], h_ref[...], c_ref[...],
                      wih_ref[...], whh_ref[...], b_ref[...],
                      gam_ref[...], bet_ref[...], None)
    oh_ref[...] = hn
    oc_ref[...] = c2


def _upd_clause(p0, p1, h, c, wih, whh, b, gamma, beta):
    return pl.pallas_call(
        _upd_clause_body,
        out_shape=[jax.ShapeDtypeStruct((NCL, DIM), jnp.float32),
                   jax.ShapeDtypeStruct((NCL, DIM), jnp.float32)],
    )(p0, p1, h, c, wih, whh, b, gamma, beta)


def _upd_lit_body(p0_ref, p1_ref, h_ref, c_ref, prev_ref,
                  wih_ref, whh_ref, b_ref, gam_ref, bet_ref,
                  oh_ref, oc_ref, chg_ref):
    h = h_ref[...]
    flipped = jnp.concatenate([h[NV:], h[:NV]], axis=0)
    hn, c2 = _lstm_bn(p0_ref[...], p1_ref[...], h, c_ref[...],
                      wih_ref[...], whh_ref[...], b_ref[...],
                      gam_ref[...], bet_ref[...], flipped)
    oh_ref[...] = hn
    oc_ref[...] = c2
    prev = prev_ref[...]
    num = jnp.sqrt(jnp.sum((hn - prev) ** 2))
    den = jnp.sqrt(jnp.sum(prev ** 2)) + 1e-6
    chg_ref[...] = (num / den).reshape(1, 1)


def _upd_lit(p0, p1, h, c, prev, wih, whh, b, gamma, beta):
    return pl.pallas_call(
        _upd_lit_body,
        out_shape=[jax.ShapeDtypeStruct((NL, DIM), jnp.float32),
                   jax.ShapeDtypeStruct((NL, DIM), jnp.float32),
                   jax.ShapeDtypeStruct((1, 1), jnp.float32)],
    )(p0, p1, h, c, prev, wih, whh, b, gamma, beta)


def _vote_body(h_ref, w0_ref, b0_ref, w1_ref, b1_ref, w2r_ref, b2_ref,
               out_ref):
    h = jnp.maximum(jnp.dot(h_ref[...], w0_ref[...],
                            preferred_element_type=jnp.float32) + b0_ref[...], 0.0)
    h = jnp.maximum(jnp.dot(h, w1_ref[...],
                            preferred_element_type=jnp.float32) + b1_ref[...], 0.0)
    votes = jnp.sum(h * w2r_ref[...], axis=1)
    pred = jnp.mean(votes) + b2_ref[0, 0]
    out_ref[...] = _sig(pred).reshape(1, 1)


def _vote(h, w0, b0, w1, b1, w2r, b2):
    return pl.pallas_call(
        _vote_body,
        out_shape=jax.ShapeDtypeStruct((1, 1), jnp.float32),
    )(h, w0, b0, w1, b1, w2r, b2)


def kernel(params, edge_index, n_lits, n_clauses, n_vars):
    p = params
    f32 = jnp.float32
    lit_idx = edge_index[0].astype(jnp.int32)
    cl_idx = edge_index[1].astype(jnp.int32)

    def lin_t(wb):
        w, b = wb
        return w.T.astype(f32), b.reshape(1, -1).astype(f32)

    def attn_parts(ps):
        (w1, b1), (w2, b2) = ps
        was = w1[:, :DIM].T.astype(f32)
        wad = w1[:, DIM:].T.astype(f32)
        ba = b1.reshape(1, -1).astype(f32)
        w2v = w2[0].astype(f32)
        b2v = jnp.full((16,), b2[0], f32)
        return was, wad, ba, w2v, b2v

    la_was, la_wad, la_ba, la_w2, la_b2 = attn_parts(p['lit_attn'])
    ca_was, ca_wad, ca_ba, ca_w2, ca_b2 = attn_parts(p['clause_attn'])
    lm = [lin_t(wb) for wb in p['lit_msg']]
    cm = [lin_t(wb) for wb in p['clause_msg']]

    def lstm_t(ps):
        wih, whh, bih, bhh = ps
        return (wih.T.astype(f32), whh.T.astype(f32),
                (bih + bhh).reshape(1, -1).astype(f32))
    l_wih, l_whh, l_b = lstm_t(p['lit_lstm'])
    c_wih, c_whh, c_b = lstm_t(p['clause_lstm'])
    l_gam = p['lit_bn'][0].reshape(1, -1).astype(f32)
    l_bet = p['lit_bn'][1].reshape(1, -1).astype(f32)
    c_gam = p['clause_bn'][0].reshape(1, -1).astype(f32)
    c_bet = p['clause_bn'][1].reshape(1, -1).astype(f32)
    v0 = lin_t(p['vote'][0])
    v1 = lin_t(p['vote'][1])
    v_w2r = p['vote'][2][0].reshape(1, -1).astype(f32)
    v_b2 = p['vote'][2][1].reshape(1, 1).astype(f32)

    sizes_zero = 0.0 * (jnp.asarray(n_lits) + jnp.asarray(n_clauses)
                        + jnp.asarray(n_vars)).astype(f32)
    lit_row = (p['lit_init'][0][:, 0] + p['lit_init'][1]).astype(f32) + sizes_zero
    cl_row = (p['clause_init'][0][:, 0] + p['clause_init'][1]).astype(f32)
    lit_h = jnp.broadcast_to(lit_row, (NL, DIM))
    clause_h = jnp.broadcast_to(cl_row, (NCL, DIM))
    lit_c = jnp.zeros((NL, DIM), f32)
    clause_c = jnp.zeros((NCL, DIM), f32)
    prev_lit_h = lit_h
    done = jnp.array(False)

    for _ in range(ROUNDS):
        g1, b1 = _prep(lit_h, clause_h, la_was, la_wad, la_ba,
                       lm[0][0], lm[0][1], lm[1][0], lm[1][1], lm[2][0], lm[2][1])
        parts1 = _edge_sc(g1, b1, lit_idx, cl_idx, la_w2, la_b2)
        clause_h_new, clause_c_new = _upd_clause(
            parts1[0], parts1[1], clause_h, clause_c,
            c_wih, c_whh, c_b, c_gam, c_bet)
        g2, b2 = _prep(clause_h_new, lit_h, ca_was, ca_wad, ca_ba,
                       cm[0][0], cm[0][1], cm[1][0], cm[1][1], cm[2][0], cm[2][1])
        parts2 = _edge_sc(g2, b2, cl_idx, lit_idx, ca_w2, ca_b2)
        lit_h_new, lit_c_new, change = _upd_lit(
            parts2[0], parts2[1], lit_h, lit_c, prev_lit_h,
            l_wih, l_whh, l_b, l_gam, l_bet)

        clause_h = jnp.where(done, clause_h, clause_h_new)
        clause_c = jnp.where(done, clause_c, clause_c_new)
        lit_c = jnp.where(done, lit_c, lit_c_new)
        lit_h = jnp.where(done, lit_h, lit_h_new)
        prev_lit_h = jnp.where(done, prev_lit_h, lit_h_new)
        done = done | (change[0, 0] < 0.01)

    return _vote(lit_h, v0[0], v0[1], v1[0], v1[1], v_w2r, v_b2)


# speedup vs baseline: 3.9297x; 1.3214x over previous
import functools

import jax
import jax.numpy as jnp
import numpy as np
from jax import lax
from jax.experimental import pallas as pl
from jax.experimental.pallas import tpu as pltpu
from jax.experimental.pallas import tpu_sc as plsc

DIM = 128
NL = 10000
NCL = 10000
NV = 5000
E = 320000
ROUNDS = 8

NCORES = 2
NSUB = 16
NW = NCORES * NSUB
CH = 64
E_PAD = -(-E // (NW * CH)) * (NW * CH)
EPW = E_PAD // NW
NCHUNK = EPW // CH
TOTCH = E_PAD // CH
RPA = 624
AGGR = 10064
OUTR = 10016

_PERM = np.concatenate([
    32 * j + np.concatenate([np.arange(0, 32, 2), np.arange(1, 32, 2)])
    for j in range(4)
])


def _sig(x):
    return 1.0 / (1.0 + jnp.exp(-x))


def _pack32(x):
    n, d = x.shape
    return lax.bitcast_convert_type(x.reshape(n, d // 2, 2), jnp.int32)


def _unpack_pair(w):
    lo = lax.bitcast_convert_type(jnp.left_shift(w, 16), jnp.float32)
    hi = lax.bitcast_convert_type(jnp.bitwise_and(w, jnp.int32(-65536)),
                                  jnp.float32)
    return lo, hi


def _lanesum16(v):
    lane = lax.iota(jnp.int32, 16)
    for sh in (8, 4, 2, 1):
        perm = jnp.bitwise_xor(lane, sh)
        v = v + v.at[perm].get(mode="promise_in_bounds")
    return v


def _edge_body(g_hbm, b_hbm, idxg_hbm, didxs_hbm, w2_hbm, b2_hbm, out_hbm,
               idxg_v0, idxg_v1, dids_v0, dids_v1, g_v0, g_v1, b_v0, b_v1,
               msg_v, w2_v, b2_v, agg_sh,
               sem_g0, sem_g1, sem_b0, sem_b1, sem_ig0, sem_ig1,
               sem_is0, sem_is1):
    c = lax.axis_index("c")
    s = lax.axis_index("s")
    wid = s * NCORES + c
    cbase = wid * NCHUNK
    ebase = wid * EPW

    idxg_v = (idxg_v0, idxg_v1)
    dids_v = (dids_v0, dids_v1)
    g_v = (g_v0, g_v1)
    b_v = (b_v0, b_v1)
    sem_g = (sem_g0, sem_g1)
    sem_b = (sem_b0, sem_b1)
    sem_ig = (sem_ig0, sem_ig1)
    sem_is = (sem_is0, sem_is1)

    pltpu.sync_copy(w2_hbm, w2_v)
    pltpu.sync_copy(b2_hbm, b2_v)

    def _zrow(i, _):
        for j in range(8):
            msg_v[i, pl.ds(16 * j, 16)] = jnp.zeros((16,), jnp.float32)
        return ()
    lax.fori_loop(0, CH, _zrow, ())
    zbase = s * RPA
    for k in range(11):
        pltpu.sync_copy(msg_v, agg_sh.at[pl.ds(zbase + CH * k, CH)])
    plsc.subcore_barrier()

    def _wait(dummy_hbm, dst, sem):
        pltpu.make_async_copy(dummy_hbm, dst, sem).wait()

    def _issue_gathers(p):
        pltpu.async_copy(g_hbm.at[idxg_v[p].at[pl.ds(0, CH)]], g_v[p],
                         sem_g[p])
        pltpu.async_copy(b_hbm.at[idxg_v[p].at[pl.ds(CH, CH)]], b_v[p],
                         sem_b[p])

    for p in range(2):
        pltpu.async_copy(idxg_hbm.at[pl.ds((cbase + p) * 2 * CH, 2 * CH)],
                         idxg_v[p], sem_ig[p])
        _wait(idxg_hbm.at[pl.ds(0, 2 * CH)], idxg_v[p], sem_ig[p])
        _issue_gathers(p)
    pltpu.async_copy(didxs_hbm.at[pl.ds(ebase, CH)], dids_v[0], sem_is[0])

    def _process(k, p):
        @pl.when(k + 1 < NCHUNK)
        def _():
            pltpu.async_copy(didxs_hbm.at[pl.ds(ebase + (k + 1) * CH, CH)],
                             dids_v[1 - p], sem_is[1 - p])

        _wait(g_hbm.at[pl.ds(0, CH)], g_v[p], sem_g[p])
        _wait(b_hbm.at[pl.ds(0, CH)], b_v[p], sem_b[p])

        @pl.when(k + 2 < NCHUNK)
        def _():
            pltpu.async_copy(
                idxg_hbm.at[pl.ds((cbase + k + 2) * 2 * CH, 2 * CH)],
                idxg_v[p], sem_ig[p])

        gp, bp = g_v[p], b_v[p]

        @plsc.parallel_loop(0, CH, unroll=4)
        def _edge(e):
            acc = jnp.zeros((16,), jnp.float32)
            for j in range(4):
                a0, a1 = _unpack_pair(gp[e, pl.ds(16 * j, 16)])
                b0 = bp[e, pl.ds(32 * j, 16)]
                b1 = bp[e, pl.ds(32 * j + 16, 16)]
                h0 = jnp.maximum(a0 + b0, 0.0)
                h1 = jnp.maximum(a1 + b1, 0.0)
                acc = (acc + h0 * w2_v[pl.ds(32 * j, 16)]
                       + h1 * w2_v[pl.ds(32 * j + 16, 16)])
            aw = _sig(_lanesum16(acc) + b2_v[...])
            for j in range(4):
                m0, m1 = _unpack_pair(gp[e, pl.ds(64 + 16 * j, 16)])
                msg_v[e, pl.ds(32 * j, 16)] = m0 * aw
                msg_v[e, pl.ds(32 * j + 16, 16)] = m1 * aw

        _wait(didxs_hbm.at[pl.ds(0, CH)], dids_v[p], sem_is[p])
        pltpu.sync_copy(msg_v, agg_sh.at[dids_v[p]], add=True)

        @pl.when(k + 2 < NCHUNK)
        def _():
            _wait(idxg_hbm.at[pl.ds(0, 2 * CH)], idxg_v[p], sem_ig[p])
            _issue_gathers(p)

    def _pair(k2, _):
        for p in range(2):
            k = 2 * k2 + p

            @pl.when(k < NCHUNK)
            def _():
                _process(k, p)
        return ()
    lax.fori_loop(0, (NCHUNK + 1) // 2, _pair, ())

    plsc.subcore_barrier()

    @pl.when(s < NSUB - 1)
    def _():
        pltpu.sync_copy(agg_sh.at[pl.ds(s * RPA, RPA)],
                        out_hbm.at[c, pl.ds(s * RPA, RPA)])

    @pl.when(s == NSUB - 1)
    def _():
        last = OUTR - (NSUB - 1) * RPA
        pltpu.sync_copy(agg_sh.at[pl.ds((NSUB - 1) * RPA, last)],
                        out_hbm.at[c, pl.ds((NSUB - 1) * RPA, last)])


_EDGE_SC_CACHE = []


def _edge_sc(*args):
    if not _EDGE_SC_CACHE:
        _EDGE_SC_CACHE.append(functools.partial(
            pl.kernel,
            mesh=plsc.VectorSubcoreMesh(core_axis_name="c", subcore_axis_name="s"),
            out_type=jax.ShapeDtypeStruct((NCORES, OUTR, DIM), jnp.float32),
            scratch_types=[
                pltpu.VMEM((2 * CH,), jnp.int32),
                pltpu.VMEM((2 * CH,), jnp.int32),
                pltpu.VMEM((CH,), jnp.int32),
                pltpu.VMEM((CH,), jnp.int32),
                pltpu.VMEM((CH, DIM), jnp.int32),
                pltpu.VMEM((CH, DIM), jnp.int32),
                pltpu.VMEM((CH, DIM), jnp.float32),
                pltpu.VMEM((CH, DIM), jnp.float32),
                pltpu.VMEM((CH, DIM), jnp.float32),
                pltpu.VMEM((DIM,), jnp.float32),
                pltpu.VMEM((16,), jnp.float32),
                pltpu.VMEM_SHARED((AGGR, DIM), jnp.float32),
                pltpu.SemaphoreType.DMA,
                pltpu.SemaphoreType.DMA,
                pltpu.SemaphoreType.DMA,
                pltpu.SemaphoreType.DMA,
                pltpu.SemaphoreType.DMA,
                pltpu.SemaphoreType.DMA,
                pltpu.SemaphoreType.DMA,
                pltpu.SemaphoreType.DMA,
            ],
        )(_edge_body))
    return _EDGE_SC_CACHE[0](*args)


def _prep_body(src_ref, dst_ref, was_ref, wad_ref, ba_ref,
               w0_ref, b0_ref, w1_ref, b1_ref, w2_ref, b2_ref,
               g_ref, b_ref):
    src = src_ref[...]
    a = jnp.dot(src, was_ref[...], preferred_element_type=jnp.float32)
    b_ref[...] = (jnp.dot(dst_ref[...], wad_ref[...],
                          preferred_element_type=jnp.float32) + ba_ref[...])
    h = jnp.maximum(jnp.dot(src, w0_ref[...],
                            preferred_element_type=jnp.float32) + b0_ref[...], 0.0)
    h = jnp.maximum(jnp.dot(h, w1_ref[...],
                            preferred_element_type=jnp.float32) + b1_ref[...], 0.0)
    m = jnp.dot(h, w2_ref[...], preferred_element_type=jnp.float32) + b2_ref[...]
    g_ref[...] = jnp.concatenate([a, m], axis=1).astype(jnp.bfloat16)


def _prep(src_h, dst_h, was, wad, ba, w0, b0, w1, b1, w2, b2):
    return pl.pallas_call(
        _prep_body,
        out_shape=[jax.ShapeDtypeStruct((NL, 2 * DIM), jnp.bfloat16),
                   jax.ShapeDtypeStruct((NL, DIM), jnp.float32)],
    )(src_h, dst_h, was, wad, ba, w0, b0, w1, b1, w2, b2)


def _lstm_bn(p0, p1, h, c, wih, whh, b, gamma, beta, extra):
    x_in = p0 + p1
    if extra is not None:
        x_in = jnp.concatenate([x_in, extra], axis=1)
    g = (jnp.dot(x_in, wih, preferred_element_type=jnp.float32)
         + jnp.dot(h, whh, preferred_element_type=jnp.float32) + b)
    i = _sig(g[:, :DIM])
    f = _sig(g[:, DIM:2 * DIM])
    gg = jnp.tanh(g[:, 2 * DIM:3 * DIM])
    o = _sig(g[:, 3 * DIM:])
    c2 = f * c + i * gg
    h2 = o * jnp.tanh(c2)
    x = h + h2
    mu = jnp.mean(x, axis=0, keepdims=True)
    var = jnp.mean((x - mu) ** 2, axis=0, keepdims=True)
    hn = (x - mu) / jnp.sqrt(var + 1e-5) * gamma + beta
    return hn, c2


def _upd_clause_body(p0_ref, p1_ref, h_ref, c_ref, wih_ref, whh_ref, b_ref,
                     gam_ref, bet_ref, oh_ref, oc_ref):
    hn, c2 = _lstm_bn(p0_ref[...], p1_ref[...], h_ref[...], c_ref[...],
                      wih_ref[...], whh_ref[...], b_ref[...],
                      gam_ref[...], bet_ref[...], None)
    oh_ref[...] = hn
    oc_ref[...] = c2


def _upd_clause(p0, p1, h, c, wih, whh, b, gamma, beta):
    return pl.pallas_call(
        _upd_clause_body,
        out_shape=[jax.ShapeDtypeStruct((NCL, DIM), jnp.float32),
                   jax.ShapeDtypeStruct((NCL, DIM), jnp.float32)],
    )(p0, p1, h, c, wih, whh, b, gamma, beta)


def _upd_lit_body(p0_ref, p1_ref, h_ref, c_ref, prev_ref,
                  wih_ref, whh_ref, b_ref, gam_ref, bet_ref,
                  oh_ref, oc_ref, chg_ref):
    h = h_ref[...]
    flipped = jnp.concatenate([h[NV:], h[:NV]], axis=0)
    hn, c2 = _lstm_bn(p0_ref[...], p1_ref[...], h, c_ref[...],
                      wih_ref[...], whh_ref[...], b_ref[...],
                      gam_ref[...], bet_ref[...], flipped)
    oh_ref[...] = hn
    oc_ref[...] = c2
    prev = prev_ref[...]
    num = jnp.sqrt(jnp.sum((hn - prev) ** 2))
    den = jnp.sqrt(jnp.sum(prev ** 2)) + 1e-6
    chg_ref[...] = (num / den).reshape(1, 1)


def _upd_lit(p0, p1, h, c, prev, wih, whh, b, gamma, beta):
    return pl.pallas_call(
        _upd_lit_body,
        out_shape=[jax.ShapeDtypeStruct((NL, DIM), jnp.float32),
                   jax.ShapeDtypeStruct((NL, DIM), jnp.float32),
                   jax.ShapeDtypeStruct((1, 1), jnp.float32)],
    )(p0, p1, h, c, prev, wih, whh, b, gamma, beta)


def _vote_body(h_ref, w0_ref, b0_ref, w1_ref, b1_ref, w2r_ref, b2_ref,
               out_ref):
    h = jnp.maximum(jnp.dot(h_ref[...], w0_ref[...],
                            preferred_element_type=jnp.float32) + b0_ref[...], 0.0)
    h = jnp.maximum(jnp.dot(h, w1_ref[...],
                            preferred_element_type=jnp.float32) + b1_ref[...], 0.0)
    votes = jnp.sum(h * w2r_ref[...], axis=1)
    pred = jnp.mean(votes) + b2_ref[0, 0]
    out_ref[...] = _sig(pred).reshape(1, 1)


def _vote(h, w0, b0, w1, b1, w2r, b2):
    return pl.pallas_call(
        _vote_body,
        out_shape=jax.ShapeDtypeStruct((1, 1), jnp.float32),
    )(h, w0, b0, w1, b1, w2r, b2)


def _edge_indices(src, dst):
    pad = E_PAD - E
    z = jnp.zeros((pad,), jnp.int32)
    src_p = jnp.concatenate([src, z])
    dstg_p = jnp.concatenate([dst, z])
    dsts_p = jnp.concatenate([dst, jnp.full((pad,), NL, jnp.int32)])
    idxg = jnp.stack([src_p.reshape(TOTCH, CH),
                      dstg_p.reshape(TOTCH, CH)], axis=1).reshape(-1)
    return idxg, dsts_p


def kernel(params, edge_index, n_lits, n_clauses, n_vars):
    p = params
    f32 = jnp.float32
    lit_idx = edge_index[0].astype(jnp.int32)
    cl_idx = edge_index[1].astype(jnp.int32)
    idxg1, didxs1 = _edge_indices(lit_idx, cl_idx)
    idxg2, didxs2 = _edge_indices(cl_idx, lit_idx)

    def lin_t(wb):
        w, b = wb
        return w.T.astype(f32), b.reshape(1, -1).astype(f32)

    def attn_parts(ps):
        (w1, b1), (w2, b2) = ps
        was = w1[:, :DIM].T.astype(f32)
        wad = w1[:, DIM:].T.astype(f32)[:, _PERM]
        ba = b1.astype(f32)[_PERM].reshape(1, -1)
        w2v = w2[0].astype(f32)[_PERM]
        b2v = jnp.full((16,), b2[0], f32)
        return was, wad, ba, w2v, b2v

    la_was, la_wad, la_ba, la_w2, la_b2 = attn_parts(p['lit_attn'])
    ca_was, ca_wad, ca_ba, ca_w2, ca_b2 = attn_parts(p['clause_attn'])
    lm = [lin_t(wb) for wb in p['lit_msg']]
    cm = [lin_t(wb) for wb in p['clause_msg']]

    def lstm_t(ps):
        wih, whh, bih, bhh = ps
        return (wih.T.astype(f32), whh.T.astype(f32),
                (bih + bhh).reshape(1, -1).astype(f32))
    l_wih, l_whh, l_b = lstm_t(p['lit_lstm'])
    c_wih, c_whh, c_b = lstm_t(p['clause_lstm'])
    c_wih = c_wih[_PERM, :]
    l_wih = jnp.concatenate([l_wih[:DIM][_PERM, :], l_wih[DIM:]], axis=0)
    l_gam = p['lit_bn'][0].reshape(1, -1).astype(f32)
    l_bet = p['lit_bn'][1].reshape(1, -1).astype(f32)
    c_gam = p['clause_bn'][0].reshape(1, -1).astype(f32)
    c_bet = p['clause_bn'][1].reshape(1, -1).astype(f32)
    v0 = lin_t(p['vote'][0])
    v1 = lin_t(p['vote'][1])
    v_w2r = p['vote'][2][0].reshape(1, -1).astype(f32)
    v_b2 = p['vote'][2][1].reshape(1, 1).astype(f32)

    sizes_zero = 0.0 * (jnp.asarray(n_lits) + jnp.asarray(n_clauses)
                        + jnp.asarray(n_vars)).astype(f32)
    lit_row = (p['lit_init'][0][:, 0] + p['lit_init'][1]).astype(f32) + sizes_zero
    cl_row = (p['clause_init'][0][:, 0] + p['clause_init'][1]).astype(f32)
    lit_h = jnp.broadcast_to(lit_row, (NL, DIM))
    clause_h = jnp.broadcast_to(cl_row, (NCL, DIM))
    lit_c = jnp.zeros((NL, DIM), f32)
    clause_c = jnp.zeros((NCL, DIM), f32)
    prev_lit_h = lit_h
    done = jnp.array(False)

    for _ in range(ROUNDS):
        g1, b1 = _prep(lit_h, clause_h, la_was, la_wad, la_ba,
                       lm[0][0], lm[0][1], lm[1][0], lm[1][1], lm[2][0], lm[2][1])
        parts1 = _edge_sc(_pack32(g1), b1, idxg1, didxs1, la_w2, la_b2)
        clause_h_new, clause_c_new = _upd_clause(
            parts1[0, :NCL], parts1[1, :NCL], clause_h, clause_c,
            c_wih, c_whh, c_b, c_gam, c_bet)
        g2, b2 = _prep(clause_h_new, lit_h, ca_was, ca_wad, ca_ba,
                       cm[0][0], cm[0][1], cm[1][0], cm[1][1], cm[2][0], cm[2][1])
        parts2 = _edge_sc(_pack32(g2), b2, idxg2, didxs2, ca_w2, ca_b2)
        lit_h_new, lit_c_new, change = _upd_lit(
            parts2[0, :NL], parts2[1, :NL], lit_h, lit_c, prev_lit_h,
            l_wih, l_whh, l_b, l_gam, l_bet)

        clause_h = jnp.where(done, clause_h, clause_h_new)
        clause_c = jnp.where(done, clause_c, clause_c_new)
        lit_c = jnp.where(done, lit_c, lit_c_new)
        lit_h = jnp.where(done, lit_h, lit_h_new)
        prev_lit_h = jnp.where(done, prev_lit_h, lit_h_new)
        done = done | (change[0, 0] < 0.01)

    return _vote(lit_h, v0[0], v0[1], v1[0], v1[1], v_w2r, v_b2)
